# Initial kernel scaffold; baseline (speedup 1.0000x reference)
import jax
import jax.numpy as jnp
from jax.experimental import pallas as pl


def kernel(objectness, pred_bbox_deltas, anchors):
    raise NotImplementedError("write your pallas kernel here")


# speedup vs baseline: 33.2010x; 33.2010x over previous
import jax
import jax.numpy as jnp
from jax.experimental import pallas as pl

_PRE_NMS_TOP_N = 2000
_POST_NMS_TOP_N = 1000
_NMS_THRESH = 0.7
_IMG_H, _IMG_W = 800.0, 800.0
_BBOX_XFORM_CLIP = 4.135166556742356
_NPAD = 2048


def _rpn_nms_kernel(scores_ref, dr_ref, ar_ref, dc_ref, ac_ref,
                    boxes_out_ref, mscores_out_ref):
    n = _NPAD
    s = scores_ref[0]
    dr = dr_ref[0]
    ar = ar_ref[0]
    dc = dc_ref[0]
    ac = ac_ref[0]

    def decode_clip(a0, a1, a2, a3, d0, d1, d2, d3):
        wa = a2 - a0
        ha = a3 - a1
        cxa = a0 + 0.5 * wa
        cya = a1 + 0.5 * ha
        dw = jnp.minimum(d2, _BBOX_XFORM_CLIP)
        dh = jnp.minimum(d3, _BBOX_XFORM_CLIP)
        pcx = d0 * wa + cxa
        pcy = d1 * ha + cya
        pw = jnp.exp(dw) * wa
        ph = jnp.exp(dh) * ha
        x1 = jnp.clip(pcx - 0.5 * pw, 0.0, _IMG_W)
        y1 = jnp.clip(pcy - 0.5 * ph, 0.0, _IMG_H)
        x2 = jnp.clip(pcx + 0.5 * pw, 0.0, _IMG_W)
        y2 = jnp.clip(pcy + 0.5 * ph, 0.0, _IMG_H)
        return x1, y1, x2, y2

    x1r, y1r, x2r, y2r = decode_clip(
        ar[0:1, :], ar[1:2, :], ar[2:3, :], ar[3:4, :],
        dr[0:1, :], dr[1:2, :], dr[2:3, :], dr[3:4, :])
    x1c, y1c, x2c, y2c = decode_clip(
        ac[:, 0:1], ac[:, 1:2], ac[:, 2:3], ac[:, 3:4],
        dc[:, 0:1], dc[:, 1:2], dc[:, 2:3], dc[:, 3:4])

    wr = x2r - x1r
    hr = y2r - y1r
    area_r = wr * hr
    area_c = (x2c - x1c) * (y2c - y1c)

    col = jax.lax.broadcasted_iota(jnp.int32, (1, n), 1)
    valid = (col < _PRE_NMS_TOP_N) & (wr >= 0.0) & (hr >= 0.0)
    valid_f = jnp.where(valid, 1.0, 0.0)

    iw = jnp.maximum(jnp.minimum(x2c, x2r) - jnp.maximum(x1c, x1r), 0.0)
    ih = jnp.maximum(jnp.minimum(y2c, y2r) - jnp.maximum(y1c, y1r), 0.0)
    inter = iw * ih
    union = area_c + area_r - inter
    iou = inter / jnp.maximum(union, 1e-9)
    ri = jax.lax.broadcasted_iota(jnp.int32, (n, n), 0)
    ci = jax.lax.broadcasted_iota(jnp.int32, (n, n), 1)
    sup_mat = jnp.where((iou > _NMS_THRESH) & (ri < ci), 1.0, 0.0)

    def cond(carry):
        _, changed = carry
        return changed

    def body(carry):
        keep, _ = carry
        nsup = jax.lax.dot(keep, sup_mat,
                           preferred_element_type=jnp.float32)
        keep_new = jnp.where(nsup > 0.0, 0.0, valid_f)
        return keep_new, jnp.any(keep_new != keep)

    keep, _ = jax.lax.while_loop(cond, body, (valid_f, True))

    boxes_out_ref[0, 0:1, :] = x1r
    boxes_out_ref[0, 1:2, :] = y1r
    boxes_out_ref[0, 2:3, :] = x2r
    boxes_out_ref[0, 3:4, :] = y2r
    mscores_out_ref[0] = jnp.where(keep > 0.0, s, -jnp.inf)


def kernel(objectness, pred_bbox_deltas, anchors):
    b, a = objectness.shape
    scores, topk_idx = jax.lax.top_k(objectness, _PRE_NMS_TOP_N)
    ag = anchors[topk_idx]
    dg = jnp.take_along_axis(pred_bbox_deltas, topk_idx[:, :, None], axis=1)

    pad = _NPAD - _PRE_NMS_TOP_N
    s_p = jnp.pad(scores, ((0, 0), (0, pad)),
                  constant_values=-jnp.inf).reshape(b, 1, _NPAD)
    ag_c = jnp.pad(ag, ((0, 0), (0, pad), (0, 0)))
    dg_c = jnp.pad(dg, ((0, 0), (0, pad), (0, 0)))
    ag_r = ag_c.transpose(0, 2, 1)
    dg_r = dg_c.transpose(0, 2, 1)

    n = _NPAD
    boxes_r, mscores = pl.pallas_call(
        _rpn_nms_kernel,
        grid=(b,),
        in_specs=[
            pl.BlockSpec((1, 1, n), lambda i: (i, 0, 0)),
            pl.BlockSpec((1, 4, n), lambda i: (i, 0, 0)),
            pl.BlockSpec((1, 4, n), lambda i: (i, 0, 0)),
            pl.BlockSpec((1, n, 4), lambda i: (i, 0, 0)),
            pl.BlockSpec((1, n, 4), lambda i: (i, 0, 0)),
        ],
        out_specs=[
            pl.BlockSpec((1, 4, n), lambda i: (i, 0, 0)),
            pl.BlockSpec((1, 1, n), lambda i: (i, 0, 0)),
        ],
        out_shape=[
            jax.ShapeDtypeStruct((b, 4, n), jnp.float32),
            jax.ShapeDtypeStruct((b, 1, n), jnp.float32),
        ],
    )(s_p, dg_r, ag_r, dg_c, ag_c)

    boxes = boxes_r[:, :, :_PRE_NMS_TOP_N].transpose(0, 2, 1)
    ms = mscores[:, 0, :_PRE_NMS_TOP_N]
    final_scores, sel = jax.lax.top_k(ms, _POST_NMS_TOP_N)
    final_boxes = jnp.take_along_axis(boxes, sel[:, :, None], axis=1)
    return final_boxes, final_scores
